# baseline (speedup 1.0000x reference)
import functools

import jax
import jax.numpy as jnp
from jax import lax
from jax.experimental import pallas as pl
from jax.experimental.pallas import tpu as pltpu
from jax.experimental.pallas import tpu_sc as plsc

B, H, W_IN, C = 1, 256, 256, 96
KS = 2
N = B * H * W_IN * C
OUT = (H * KS) * (W_IN * KS) * C
NC, NS, L = 2, 16, 16
NWIN = 32
WIN = OUT // NWIN
PASSES = NWIN // NC
CHUNK = 16384
SHARE = N // NS
NCHUNK = SHARE // CHUNK
NPAIR = NCHUNK // 2
WSLICE = WIN // NS
NWB = WSLICE // CHUNK


def _unpool_sc(ind_flat, pool_flat):
    mesh = plsc.VectorSubcoreMesh(core_axis_name="c", subcore_axis_name="s")

    @functools.partial(
        pl.kernel,
        mesh=mesh,
        out_type=jax.ShapeDtypeStruct((OUT,), jnp.float32),
        scratch_types=[
            pltpu.VMEM((CHUNK,), jnp.int32),
            pltpu.VMEM((CHUNK,), jnp.float32),
            pltpu.VMEM((CHUNK,), jnp.int32),
            pltpu.VMEM((CHUNK,), jnp.float32),
            pltpu.VMEM((CHUNK,), jnp.float32),
            pltpu.VMEM_SHARED((WIN,), jnp.float32),
        ],
    )
    def k(ind_hbm, pool_hbm, out_hbm,
          idx_v, val_v, tidx_v, tval_v, zero_v, win_sh):
        c = lax.axis_index("c")
        s = lax.axis_index("s")
        lanes = lax.iota(jnp.int32, L)

        def zb(i, carry):
            zero_v[pl.ds(i * L, L)] = jnp.zeros((L,), jnp.float32)
            return carry
        lax.fori_loop(0, CHUNK // L, zb, None)

        def do_pass(p, carry):
            lo = (p * NC + c) * WIN

            def zwin(j, cy):
                pltpu.sync_copy(zero_v,
                                win_sh.at[pl.ds(s * WSLICE + j * CHUNK, CHUNK)])
                return cy
            lax.fori_loop(0, NWB, zwin, None)
            plsc.subcore_barrier()

            def do_chunk(kk, cy):
                base = s * SHARE + kk * CHUNK
                pltpu.sync_copy(ind_hbm.at[pl.ds(base, CHUNK)], idx_v)
                pltpu.sync_copy(pool_hbm.at[pl.ds(base, CHUNK)], val_v)

                def vec(i, cz):
                    for u in range(4):
                        o = i * (4 * L) + u * L
                        rel = idx_v[pl.ds(o, L)] - lo
                        vv = val_v[pl.ds(o, L)]
                        ok = (rel >= 0) & (rel < WIN)
                        pad = (s * CHUNK + o) + lanes
                        tidx_v[pl.ds(o, L)] = jnp.where(ok, rel, pad)
                        tval_v[pl.ds(o, L)] = jnp.where(ok, vv, 0.0)
                    return cz
                lax.fori_loop(0, CHUNK // (4 * L), vec, None)
                pltpu.sync_copy(tval_v, win_sh.at[tidx_v], add=True)
                return cy
            lax.fori_loop(0, NCHUNK, do_chunk, None)
            plsc.subcore_barrier()

            def wb(j, cy):
                o = s * WSLICE + j * CHUNK
                pltpu.sync_copy(win_sh.at[pl.ds(o, CHUNK)],
                                out_hbm.at[pl.ds(lo + o, CHUNK)])
                return cy
            lax.fori_loop(0, NWB, wb, None)
            plsc.subcore_barrier()
            return carry
        lax.fori_loop(0, PASSES, do_pass, None)

    return k(ind_flat, pool_flat)


def kernel(pool, ind, k_size):
    pool_flat = pool.reshape(N)
    ind_flat = ind.reshape(N) + (jnp.asarray(k_size, jnp.int32) - KS)
    out = _unpool_sc(ind_flat, pool_flat)
    return out.reshape(B, H * KS, W_IN * KS, C)


# speedup vs baseline: 1.2417x; 1.2417x over previous
import functools

import jax
import jax.numpy as jnp
from jax import lax
from jax.experimental import pallas as pl
from jax.experimental.pallas import tpu as pltpu
from jax.experimental.pallas import tpu_sc as plsc

B, H, W_IN, C = 1, 256, 256, 96
KS = 2
N = B * H * W_IN * C
OUT = (H * KS) * (W_IN * KS) * C
NC, NS, L = 2, 16, 16
NWIN = 32
WIN = OUT // NWIN
PASSES = NWIN // NC
CHUNK = 16384
SHARE = N // NS
NCHUNK = SHARE // CHUNK
NPAIR = NCHUNK // 2
WSLICE = WIN // NS
NWB = WSLICE // CHUNK


def _unpool_sc(ind_flat, pool_ext):
    mesh = plsc.VectorSubcoreMesh(core_axis_name="c", subcore_axis_name="s")

    @functools.partial(
        pl.kernel,
        mesh=mesh,
        out_type=jax.ShapeDtypeStruct((OUT,), jnp.float32),
        scratch_types=[
            pltpu.VMEM((CHUNK,), jnp.int32),
            pltpu.VMEM((CHUNK,), jnp.int32),
            pltpu.VMEM((CHUNK,), jnp.float32),
            pltpu.VMEM((CHUNK,), jnp.int32),
            pltpu.VMEM((CHUNK,), jnp.float32),
            pltpu.VMEM_SHARED((WIN,), jnp.float32),
            pltpu.SemaphoreType.DMA,
            pltpu.SemaphoreType.DMA,
        ],
    )
    def k(ind_hbm, pool_hbm, out_hbm,
          idx0, idx1, val_v, tidx_v, tval_v, win_sh,
          lsem0, lsem1):
        c = lax.axis_index("c")
        s = lax.axis_index("s")
        lanes = lax.iota(jnp.int32, L)
        idxb = (idx0, idx1)
        lsem = (lsem0, lsem1)

        def start_load(kk, b):
            base = s * SHARE + kk * CHUNK
            pltpu.async_copy(ind_hbm.at[pl.ds(base, CHUNK)], idxb[b], lsem[b])

        def wait_load(kk, b):
            base = s * SHARE + kk * CHUNK
            pltpu.make_async_copy(ind_hbm.at[pl.ds(base, CHUNK)],
                                  idxb[b], lsem[b]).wait()

        def do_pass(p, carry):
            lo = (p * NC + c) * WIN

            def zwin(j, cy):
                pltpu.sync_copy(pool_hbm.at[pl.ds(N, CHUNK)],
                                win_sh.at[pl.ds(s * WSLICE + j * CHUNK, CHUNK)])
                return cy
            lax.fori_loop(0, NWB, zwin, None)
            plsc.subcore_barrier()

            start_load(0, 0)
            start_load(1, 1)

            def do_chunk(kk2, cy):
                for b in (0, 1):
                    kk = kk2 * 2 + b
                    wait_load(kk, b)

                    @pl.when(kk + 2 < NCHUNK)
                    def _():
                        start_load(kk + 2, b)

                    base2 = s * SHARE + kk * CHUNK
                    pltpu.sync_copy(pool_hbm.at[pl.ds(base2, CHUNK)], val_v)
                    ib = idxb[b]
                    vb = val_v

                    def vec(i, cz):
                        o = i * L
                        rel = ib[pl.ds(o, L)] - lo
                        vv = vb[pl.ds(o, L)]
                        ok = (rel >= 0) & (rel < WIN)
                        pad = (s * CHUNK + o) + lanes
                        tidx_v[pl.ds(o, L)] = jnp.where(ok, rel, pad)
                        tval_v[pl.ds(o, L)] = jnp.where(ok, vv, 0.0)
                        return cz
                    lax.fori_loop(0, CHUNK // L, vec, None)
                    pltpu.sync_copy(tval_v, win_sh.at[tidx_v], add=True)
                return cy
            lax.fori_loop(0, NPAIR, do_chunk, None)
            plsc.subcore_barrier()

            def wb(j, cy):
                o = s * WSLICE + j * CHUNK
                pltpu.sync_copy(win_sh.at[pl.ds(o, CHUNK)],
                                out_hbm.at[pl.ds(lo + o, CHUNK)])
                return cy
            lax.fori_loop(0, NWB, wb, None)
            plsc.subcore_barrier()
            return carry
        lax.fori_loop(0, PASSES, do_pass, None)

    return k(ind_flat, pool_ext)


def kernel(pool, ind, k_size):
    pool_flat = pool.reshape(N)
    ind_flat = ind.reshape(N) + (jnp.asarray(k_size, jnp.int32) - KS)
    pool_ext = jnp.concatenate([pool_flat, jnp.zeros((CHUNK,), jnp.float32)])
    out = _unpool_sc(ind_flat, pool_ext)
    return out.reshape(B, H * KS, W_IN * KS, C)


# speedup vs baseline: 1.2992x; 1.0463x over previous
import functools

import jax
import jax.numpy as jnp
from jax import lax
from jax.experimental import pallas as pl
from jax.experimental.pallas import tpu as pltpu
from jax.experimental.pallas import tpu_sc as plsc

B, H, W_IN, C = 1, 256, 256, 96
KS = 2
N = B * H * W_IN * C
OUT = (H * KS) * (W_IN * KS) * C
NC, NS, L = 2, 16, 16
NWIN = 32
WIN = OUT // NWIN
PASSES = NWIN // NC
CHUNK = 16384
SHARE = N // NS
NCHUNK = SHARE // CHUNK
NPAIR = NCHUNK // 2
WSLICE = WIN // NS
NWB = WSLICE // CHUNK


def _unpool_sc(ind_flat, pool_ext):
    mesh = plsc.VectorSubcoreMesh(core_axis_name="c", subcore_axis_name="s")

    @functools.partial(
        pl.kernel,
        mesh=mesh,
        out_type=jax.ShapeDtypeStruct((OUT,), jnp.float32),
        scratch_types=[
            pltpu.VMEM((CHUNK,), jnp.int32),
            pltpu.VMEM((CHUNK,), jnp.int32),
            pltpu.VMEM((CHUNK,), jnp.float32),
            pltpu.VMEM((CHUNK,), jnp.int32),
            pltpu.VMEM((CHUNK,), jnp.float32),
            pltpu.VMEM_SHARED((WIN,), jnp.float32),
            pltpu.SemaphoreType.DMA,
            pltpu.SemaphoreType.DMA,
        ],
    )
    def k(ind_hbm, pool_hbm, out_hbm,
          idx0, idx1, val_v, tidx_v, tval_v, win_sh,
          lsem0, lsem1):
        c = lax.axis_index("c")
        s = lax.axis_index("s")
        lanes = lax.iota(jnp.int32, L)
        idxb = (idx0, idx1)
        lsem = (lsem0, lsem1)

        def start_load(kk, b):
            base = s * SHARE + kk * CHUNK
            pltpu.async_copy(ind_hbm.at[pl.ds(base, CHUNK)], idxb[b], lsem[b])

        def wait_load(kk, b):
            base = s * SHARE + kk * CHUNK
            pltpu.make_async_copy(ind_hbm.at[pl.ds(base, CHUNK)],
                                  idxb[b], lsem[b]).wait()

        def do_pass(p, carry):
            lo = (p * NC + c) * WIN

            def zwin(j, cy):
                pltpu.sync_copy(pool_hbm.at[pl.ds(N, CHUNK)],
                                win_sh.at[pl.ds(s * WSLICE + j * CHUNK, CHUNK)])
                return cy
            lax.fori_loop(0, NWB, zwin, None)
            plsc.subcore_barrier()

            start_load(0, 0)
            start_load(1, 1)

            def do_chunk(kk2, cy):
                for b in (0, 1):
                    kk = kk2 * 2 + b
                    wait_load(kk, b)
                    base2 = s * SHARE + kk * CHUNK
                    pltpu.sync_copy(pool_hbm.at[pl.ds(base2, CHUNK)], val_v)
                    ib = idxb[b]
                    vb = val_v

                    def vec(i, cz):
                        o = i * L
                        rel = ib[pl.ds(o, L)] - lo
                        vv = vb[pl.ds(o, L)]
                        ok = (rel >= 0) & (rel < WIN)
                        pad = (s * CHUNK + o) + lanes
                        tidx_v[pl.ds(o, L)] = jnp.where(ok, rel, pad)
                        tval_v[pl.ds(o, L)] = jnp.where(ok, vv, 0.0)
                        return cz
                    lax.fori_loop(0, CHUNK // L, vec, None)

                    @pl.when(kk + 2 < NCHUNK)
                    def _():
                        start_load(kk + 2, b)
                    pltpu.sync_copy(tval_v, win_sh.at[tidx_v], add=True)
                return cy
            lax.fori_loop(0, NPAIR, do_chunk, None)
            plsc.subcore_barrier()

            def wb(j, cy):
                o = s * WSLICE + j * CHUNK
                pltpu.sync_copy(win_sh.at[pl.ds(o, CHUNK)],
                                out_hbm.at[pl.ds(lo + o, CHUNK)])
                return cy
            lax.fori_loop(0, NWB, wb, None)
            plsc.subcore_barrier()
            return carry
        lax.fori_loop(0, PASSES, do_pass, None)

    return k(ind_flat, pool_ext)


def kernel(pool, ind, k_size):
    pool_flat = pool.reshape(N)
    ind_flat = ind.reshape(N) + (jnp.asarray(k_size, jnp.int32) - KS)
    pool_ext = jnp.concatenate([pool_flat, jnp.zeros((CHUNK,), jnp.float32)])
    out = _unpool_sc(ind_flat, pool_ext)
    return out.reshape(B, H * KS, W_IN * KS, C)


# speedup vs baseline: 1.4619x; 1.1252x over previous
import functools

import jax
import jax.numpy as jnp
from jax import lax
from jax.experimental import pallas as pl
from jax.experimental.pallas import tpu as pltpu
from jax.experimental.pallas import tpu_sc as plsc

B, H, W_IN, C = 1, 256, 256, 96
KS = 2
N = B * H * W_IN * C
OUT = (H * KS) * (W_IN * KS) * C
NC, NS, L = 2, 16, 16
NWIN = 32
WIN = OUT // NWIN
PASSES = NWIN // NC
CHUNK = 16384
SHARE = N // NS
NCHUNK = SHARE // CHUNK
NPAIR = NCHUNK // 2
WSLICE = WIN // NS
NWB = WSLICE // CHUNK


def _unpool_sc(ind_flat, pool_ext):
    mesh = plsc.VectorSubcoreMesh(core_axis_name="c", subcore_axis_name="s")

    @functools.partial(
        pl.kernel,
        mesh=mesh,
        out_type=jax.ShapeDtypeStruct((OUT,), jnp.float32),
        scratch_types=[
            pltpu.VMEM((CHUNK,), jnp.int32),
            pltpu.VMEM((CHUNK,), jnp.int32),
            pltpu.VMEM((CHUNK,), jnp.float32),
            pltpu.VMEM((CHUNK,), jnp.int32),
            pltpu.VMEM((CHUNK,), jnp.float32),
            pltpu.VMEM_SHARED((WIN,), jnp.float32),
            pltpu.SemaphoreType.DMA,
            pltpu.SemaphoreType.DMA,
            pltpu.SemaphoreType.DMA,
        ],
    )
    def k(ind_hbm, pool_hbm, out_hbm,
          idx0, idx1, val_v, tidx_v, tval_v, win_sh,
          lsem0, lsem1, psem):
        c = lax.axis_index("c")
        s = lax.axis_index("s")
        lanes = lax.iota(jnp.int32, L)
        idxb = (idx0, idx1)
        lsem = (lsem0, lsem1)

        def start_load(kk, b):
            base = s * SHARE + kk * CHUNK
            pltpu.async_copy(ind_hbm.at[pl.ds(base, CHUNK)], idxb[b], lsem[b])

        def wait_load(kk, b):
            base = s * SHARE + kk * CHUNK
            pltpu.make_async_copy(ind_hbm.at[pl.ds(base, CHUNK)],
                                  idxb[b], lsem[b]).wait()

        def start_pool(kk):
            base = s * SHARE + kk * CHUNK
            pltpu.async_copy(pool_hbm.at[pl.ds(base, CHUNK)], val_v, psem)

        def wait_pool(kk):
            base = s * SHARE + kk * CHUNK
            pltpu.make_async_copy(pool_hbm.at[pl.ds(base, CHUNK)],
                                  val_v, psem).wait()

        def do_pass(p, carry):
            lo = (p * NC + c) * WIN

            def zwin(j, cy):
                pltpu.sync_copy(pool_hbm.at[pl.ds(N, CHUNK)],
                                win_sh.at[pl.ds(s * WSLICE + j * CHUNK, CHUNK)])
                return cy
            lax.fori_loop(0, NWB, zwin, None)
            plsc.subcore_barrier()

            start_load(0, 0)
            start_load(1, 1)
            start_pool(0)

            def do_chunk(kk2, cy):
                for b in (0, 1):
                    kk = kk2 * 2 + b
                    wait_load(kk, b)
                    wait_pool(kk)
                    ib = idxb[b]
                    vb = val_v

                    def vec(i, cz):
                        o = i * L
                        rel = ib[pl.ds(o, L)] - lo
                        vv = vb[pl.ds(o, L)]
                        ok = (rel >= 0) & (rel < WIN)
                        pad = (s * CHUNK + o) + lanes
                        tidx_v[pl.ds(o, L)] = jnp.where(ok, rel, pad)
                        tval_v[pl.ds(o, L)] = jnp.where(ok, vv, 0.0)
                        return cz
                    lax.fori_loop(0, CHUNK // L, vec, None)

                    @pl.when(kk + 2 < NCHUNK)
                    def _():
                        start_load(kk + 2, b)

                    @pl.when(kk + 1 < NCHUNK)
                    def _():
                        start_pool(kk + 1)
                    pltpu.sync_copy(tval_v, win_sh.at[tidx_v], add=True)
                return cy
            lax.fori_loop(0, NPAIR, do_chunk, None)
            plsc.subcore_barrier()

            def wb(j, cy):
                o = s * WSLICE + j * CHUNK
                pltpu.sync_copy(win_sh.at[pl.ds(o, CHUNK)],
                                out_hbm.at[pl.ds(lo + o, CHUNK)])
                return cy
            lax.fori_loop(0, NWB, wb, None)
            plsc.subcore_barrier()
            return carry
        lax.fori_loop(0, PASSES, do_pass, None)

    return k(ind_flat, pool_ext)


def kernel(pool, ind, k_size):
    pool_flat = pool.reshape(N)
    ind_flat = ind.reshape(N) + (jnp.asarray(k_size, jnp.int32) - KS)
    pool_ext = jnp.concatenate([pool_flat, jnp.zeros((CHUNK,), jnp.float32)])
    out = _unpool_sc(ind_flat, pool_ext)
    return out.reshape(B, H * KS, W_IN * KS, C)


# speedup vs baseline: 1.4802x; 1.0125x over previous
import functools

import jax
import jax.numpy as jnp
from jax import lax
from jax.experimental import pallas as pl
from jax.experimental.pallas import tpu as pltpu
from jax.experimental.pallas import tpu_sc as plsc

B, H, W_IN, C = 1, 256, 256, 96
KS = 2
N = B * H * W_IN * C
OUT = (H * KS) * (W_IN * KS) * C
NC, NS, L = 2, 16, 16
NWIN = 32
WIN = OUT // NWIN
PASSES = NWIN // NC
CHUNK = 16384
SHARE = N // NS
NCHUNK = SHARE // CHUNK
NPAIR = NCHUNK // 2
WSLICE = WIN // NS
NWB = 1


def _unpool_sc(ind_flat, pool_ext):
    mesh = plsc.VectorSubcoreMesh(core_axis_name="c", subcore_axis_name="s")

    @functools.partial(
        pl.kernel,
        mesh=mesh,
        out_type=jax.ShapeDtypeStruct((OUT,), jnp.float32),
        scratch_types=[
            pltpu.VMEM((CHUNK,), jnp.int32),
            pltpu.VMEM((CHUNK,), jnp.int32),
            pltpu.VMEM((CHUNK,), jnp.float32),
            pltpu.VMEM((CHUNK,), jnp.int32),
            pltpu.VMEM((CHUNK,), jnp.float32),
            pltpu.VMEM_SHARED((WIN,), jnp.float32),
            pltpu.SemaphoreType.DMA,
            pltpu.SemaphoreType.DMA,
            pltpu.SemaphoreType.DMA,
        ],
    )
    def k(ind_hbm, pool_hbm, out_hbm,
          idx0, idx1, val_v, tidx_v, tval_v, win_sh,
          lsem0, lsem1, psem):
        c = lax.axis_index("c")
        s = lax.axis_index("s")
        lanes = lax.iota(jnp.int32, L)
        idxb = (idx0, idx1)
        lsem = (lsem0, lsem1)

        def start_load(kk, b):
            base = s * SHARE + kk * CHUNK
            pltpu.async_copy(ind_hbm.at[pl.ds(base, CHUNK)], idxb[b], lsem[b])

        def wait_load(kk, b):
            base = s * SHARE + kk * CHUNK
            pltpu.make_async_copy(ind_hbm.at[pl.ds(base, CHUNK)],
                                  idxb[b], lsem[b]).wait()

        def start_pool(kk):
            base = s * SHARE + kk * CHUNK
            pltpu.async_copy(pool_hbm.at[pl.ds(base, CHUNK)], val_v, psem)

        def wait_pool(kk):
            base = s * SHARE + kk * CHUNK
            pltpu.make_async_copy(pool_hbm.at[pl.ds(base, CHUNK)],
                                  val_v, psem).wait()

        def do_pass(p, carry):
            lo = (p * NC + c) * WIN

            pltpu.sync_copy(pool_hbm.at[pl.ds(N, WSLICE)],
                            win_sh.at[pl.ds(s * WSLICE, WSLICE)])
            plsc.subcore_barrier()

            start_load(0, 0)
            start_load(1, 1)
            start_pool(0)

            def do_chunk(kk2, cy):
                for b in (0, 1):
                    kk = kk2 * 2 + b
                    wait_load(kk, b)
                    wait_pool(kk)
                    ib = idxb[b]
                    vb = val_v

                    def vec(i, cz):
                        o = i * L
                        rel = ib[pl.ds(o, L)] - lo
                        vv = vb[pl.ds(o, L)]
                        ok = (rel >= 0) & (rel < WIN)
                        pad = (s * CHUNK + o) + lanes
                        tidx_v[pl.ds(o, L)] = jnp.where(ok, rel, pad)
                        tval_v[pl.ds(o, L)] = jnp.where(ok, vv, 0.0)
                        return cz
                    lax.fori_loop(0, CHUNK // L, vec, None)

                    @pl.when(kk + 2 < NCHUNK)
                    def _():
                        start_load(kk + 2, b)

                    @pl.when(kk + 1 < NCHUNK)
                    def _():
                        start_pool(kk + 1)
                    pltpu.sync_copy(tval_v, win_sh.at[tidx_v], add=True)
                return cy
            lax.fori_loop(0, NPAIR, do_chunk, None)
            plsc.subcore_barrier()

            o = s * WSLICE
            pltpu.sync_copy(win_sh.at[pl.ds(o, WSLICE)],
                            out_hbm.at[pl.ds(lo + o, WSLICE)])
            plsc.subcore_barrier()
            return carry
        lax.fori_loop(0, PASSES, do_pass, None)

    return k(ind_flat, pool_ext)


def kernel(pool, ind, k_size):
    pool_flat = pool.reshape(N)
    ind_flat = ind.reshape(N) + (jnp.asarray(k_size, jnp.int32) - KS)
    pool_ext = jnp.concatenate([pool_flat, jnp.zeros((WSLICE,), jnp.float32)])
    out = _unpool_sc(ind_flat, pool_ext)
    return out.reshape(B, H * KS, W_IN * KS, C)


# speedup vs baseline: 1.8598x; 1.2565x over previous
import functools

import jax
import jax.numpy as jnp
from jax import lax
from jax.experimental import pallas as pl
from jax.experimental.pallas import tpu as pltpu
from jax.experimental.pallas import tpu_sc as plsc

B, H, W_IN, C = 1, 256, 256, 96
KS = 2
N = B * H * W_IN * C
OUT = (H * KS) * (W_IN * KS) * C
NC, NS, L = 2, 16, 16
NWIN = 32
WIN = OUT // NWIN
PASSES = NWIN // NC
CHUNK = 8192
SHARE = N // NS
NCHUNK = SHARE // CHUNK
NPAIR = NCHUNK // 2
WSLICE = WIN // NS


def _unpool_sc(packed_ext):
    mesh = plsc.VectorSubcoreMesh(core_axis_name="c", subcore_axis_name="s")

    @functools.partial(
        pl.kernel,
        mesh=mesh,
        out_type=jax.ShapeDtypeStruct((OUT,), jnp.float32),
        scratch_types=[
            pltpu.VMEM((2 * CHUNK,), jnp.float32),
            pltpu.VMEM((2 * CHUNK,), jnp.float32),
            pltpu.VMEM((CHUNK,), jnp.int32),
            pltpu.VMEM((CHUNK,), jnp.int32),
            pltpu.VMEM((CHUNK,), jnp.float32),
            pltpu.VMEM((CHUNK,), jnp.float32),
            pltpu.VMEM_SHARED((WIN,), jnp.float32),
            pltpu.SemaphoreType.DMA,
            pltpu.SemaphoreType.DMA,
            pltpu.SemaphoreType.DMA,
            pltpu.SemaphoreType.DMA,
        ],
    )
    def k(pk_hbm, out_hbm,
          pb0, pb1, si0, si1, sv0, sv1, win_sh,
          lsem0, lsem1, ssem0, ssem1):
        c = lax.axis_index("c")
        s = lax.axis_index("s")
        lanes = lax.iota(jnp.int32, L)
        pb = (pb0, pb1)
        sib = (si0, si1)
        svb = (sv0, sv1)
        lsem = (lsem0, lsem1)
        ssem = (ssem0, ssem1)

        def start_load(kk, b):
            base = (s * NCHUNK + kk) * (2 * CHUNK)
            pltpu.async_copy(pk_hbm.at[pl.ds(base, 2 * CHUNK)], pb[b], lsem[b])

        def wait_load(kk, b):
            base = (s * NCHUNK + kk) * (2 * CHUNK)
            pltpu.make_async_copy(pk_hbm.at[pl.ds(base, 2 * CHUNK)],
                                  pb[b], lsem[b]).wait()

        def start_scat(b):
            pltpu.async_copy(svb[b], win_sh.at[sib[b]], ssem[b])

        def wait_scat(b):
            pltpu.make_async_copy(svb[b], win_sh.at[sib[b]], ssem[b]).wait()

        def do_pass(p, carry):
            lo = (p * NC + c) * WIN

            pltpu.sync_copy(pk_hbm.at[pl.ds(2 * N, WSLICE)],
                            win_sh.at[pl.ds(s * WSLICE, WSLICE)])
            plsc.subcore_barrier()

            start_load(0, 0)
            start_load(1, 1)

            def do_chunk(kk2, cy):
                for b in (0, 1):
                    kk = kk2 * 2 + b
                    wait_load(kk, b)

                    @pl.when(kk >= 2)
                    def _():
                        wait_scat(b)

                    ib = pb[b]
                    si = sib[b]
                    sv = svb[b]

                    def vec(i, cz):
                        o = i * L
                        rel = jax.lax.bitcast_convert_type(ib[pl.ds(o, L)], jnp.int32) - lo
                        vv = ib[pl.ds(CHUNK + o, L)]
                        ok = (rel >= 0) & (rel < WIN)
                        pad = (s * CHUNK + o) + lanes
                        si[pl.ds(o, L)] = jnp.where(ok, rel, pad)
                        sv[pl.ds(o, L)] = jnp.where(ok, vv, 0.0)
                        return cz
                    lax.fori_loop(0, CHUNK // L, vec, None)

                    @pl.when(kk + 2 < NCHUNK)
                    def _():
                        start_load(kk + 2, b)
                    start_scat(b)
                return cy
            lax.fori_loop(0, NPAIR, do_chunk, None)
            wait_scat(0)
            wait_scat(1)
            plsc.subcore_barrier()

            o = s * WSLICE
            pltpu.sync_copy(win_sh.at[pl.ds(o, WSLICE)],
                            out_hbm.at[pl.ds(lo + o, WSLICE)])
            plsc.subcore_barrier()
            return carry
        lax.fori_loop(0, PASSES, do_pass, None)

    return k(packed_ext)


def kernel(pool, ind, k_size):
    pool_flat = pool.reshape(N)
    ind_flat = ind.reshape(N) + (jnp.asarray(k_size, jnp.int32) - KS)
    ind_f = jax.lax.bitcast_convert_type(ind_flat, jnp.float32)
    packed = jnp.stack([ind_f.reshape(-1, CHUNK),
                        pool_flat.reshape(-1, CHUNK)], axis=1).reshape(-1)
    packed_ext = jnp.concatenate([packed, jnp.zeros((WSLICE,), jnp.float32)])
    out = _unpool_sc(packed_ext)
    return out.reshape(B, H * KS, W_IN * KS, C)
